# baseline (speedup 1.0000x reference)
import functools

import jax
import jax.numpy as jnp
from jax import lax
from jax.experimental import pallas as pl
from jax.experimental.pallas import tpu as pltpu
from jax.experimental.pallas import tpu_sc as plsc

N = 10000
NP = 10240
D = 128
DH = 256
E = 320000
HCH = 128
HROWS = 2560
CHUNK = 64
EROWS = 5120
EPAD = 327680
NC, NS = 2, 16
NW = NC * NS
HRPT = HROWS // NW
RPT = EROWS // NW
HIB = 8
IB = 16
NBUF = 3
F0 = 1024
ZROWS = NP // NS

@functools.cache
def _mesh():
    return plsc.VectorSubcoreMesh(
        core_axis_name="c", subcore_axis_name="s", num_cores=NC, num_subcores=NS
    )


def _zero_vmem(ref):
    rows, cols = ref.shape

    @pl.loop(0, rows)
    def _(r):
        @pl.loop(0, cols, step=16)
        def _(c):
            ref[r, pl.ds(c, 16)] = jnp.zeros((16,), ref.dtype)


def _sc_hist(dsta, dstb):
    @functools.partial(
        pl.kernel,
        out_type=jax.ShapeDtypeStruct((NC, NP, D), jnp.float32),
        mesh=_mesh(),
        scratch_types=[
            pltpu.VMEM((HIB, HCH), jnp.int32),
            pltpu.VMEM((HCH, D), jnp.float32),
            pltpu.VMEM((HCH, D), jnp.float32),
            pltpu.VMEM((64, D), jnp.float32),
            pltpu.VMEM_SHARED((NP, D), jnp.float32),
        ],
    )
    def k(da_hbm, db_hbm, o_hbm, dstv, onea, oneb, zv, h):
        cid = lax.axis_index("c")
        sid = lax.axis_index("s")
        wid = cid * NS + sid
        e0 = jnp.where(
            lax.iota(jnp.int32, 16) == 0, jnp.float32(1.0), jnp.float32(0.0)
        )

        _zero_vmem(onea)
        _zero_vmem(oneb)
        _zero_vmem(zv)

        @pl.loop(0, HCH)
        def _(r):
            onea[r, pl.ds(0, 16)] = e0
            oneb[r, pl.ds(16, 16)] = e0

        @pl.loop(0, ZROWS, step=64)
        def _(i):
            pltpu.sync_copy(zv, h.at[pl.ds(sid * ZROWS + i, 64)])

        plsc.subcore_barrier()

        @pl.loop(0, HRPT, step=HIB)
        def _(base):
            pltpu.sync_copy(da_hbm.at[pl.ds(wid * HRPT + base, HIB)], dstv)

            @pl.loop(0, HIB)
            def _(j):
                pltpu.sync_copy(onea, h.at[dstv.at[j]], add=True)

        @pl.loop(0, HRPT, step=HIB)
        def _(base):
            pltpu.sync_copy(db_hbm.at[pl.ds(wid * HRPT + base, HIB)], dstv)

            @pl.loop(0, HIB)
            def _(j):
                pltpu.sync_copy(oneb, h.at[dstv.at[j]], add=True)

        plsc.subcore_barrier()
        pltpu.sync_copy(
            h.at[pl.ds(sid * ZROWS, ZROWS)],
            o_hbm.at[cid, pl.ds(sid * ZROWS, ZROWS)],
        )

    return k(dsta, dstb)


def _sc_aggregate(y, src2d, dst2d):
    @functools.partial(
        pl.kernel,
        out_type=jax.ShapeDtypeStruct((NC, NP, D), jnp.float32),
        mesh=_mesh(),
        scratch_types=[
            pltpu.VMEM((IB, CHUNK), jnp.int32),
            pltpu.VMEM((IB, CHUNK), jnp.int32),
            pltpu.VMEM((CHUNK, D), jnp.float32),
            pltpu.VMEM((CHUNK, D), jnp.float32),
            pltpu.VMEM((CHUNK, D), jnp.float32),
            pltpu.VMEM_SHARED((NP, D), jnp.float32),
            pltpu.SemaphoreType.DMA,
            pltpu.SemaphoreType.DMA,
            pltpu.SemaphoreType.DMA,
        ],
    )
    def k(y_hbm, src_hbm, dst_hbm, out_hbm,
          srcv, dstv, rb0, rb1, rb2, accum, s0, s1, s2):
        cid = lax.axis_index("c")
        sid = lax.axis_index("s")
        rbs = (rb0, rb1, rb2)
        sems = (s0, s1, s2)
        base = jnp.where(cid == 0, sid * (F0 // NS),
                         F0 + sid * ((EROWS - F0) // NS))
        nrows = jnp.where(cid == 0, F0 // NS, (EROWS - F0) // NS)

        _zero_vmem(rb0)

        @pl.loop(0, ZROWS, step=CHUNK)
        def _(i):
            pltpu.sync_copy(rb0, accum.at[pl.ds(sid * ZROWS + i, CHUNK)])

        plsc.subcore_barrier()

        @pl.loop(0, nrows, step=IB)
        def _(r):
            pltpu.sync_copy(src_hbm.at[pl.ds(base + r, IB)], srcv)
            pltpu.sync_copy(dst_hbm.at[pl.ds(base + r, IB)], dstv)
            for j in range(NBUF - 1):
                pltpu.async_copy(y_hbm.at[srcv.at[j]], rbs[j], sems[j])
            for j in range(IB):
                b = j % NBUF
                pltpu.make_async_copy(y_hbm.at[srcv.at[j]], rbs[b], sems[b]).wait()
                jn = j + NBUF - 1
                if jn < IB:
                    bn = jn % NBUF
                    pltpu.async_copy(y_hbm.at[srcv.at[jn]], rbs[bn], sems[bn])
                pltpu.sync_copy(rbs[b], accum.at[dstv.at[j]], add=True)

        plsc.subcore_barrier()
        pltpu.sync_copy(
            accum.at[pl.ds(sid * ZROWS, ZROWS)],
            out_hbm.at[cid, pl.ds(sid * ZROWS, ZROWS)],
        )

    return k(y, src2d, dst2d)


_BLK = 1024


def _dis_block(h, c):
    deg = 1.0 + h[0, :, c:c + 1] + h[1, :, c:c + 1]
    return lax.rsqrt(deg)


def _tc_scale(x, hist):
    def body(x_ref, h_ref, o_ref):
        o_ref[...] = x_ref[...] * _dis_block(h_ref[...], 0)

    return pl.pallas_call(
        body,
        grid=(NP // _BLK,),
        in_specs=[
            pl.BlockSpec((_BLK, D), lambda i: (i, 0)),
            pl.BlockSpec((NC, _BLK, D), lambda i: (0, i, 0)),
        ],
        out_specs=pl.BlockSpec((_BLK, D), lambda i: (i, 0)),
        out_shape=jax.ShapeDtypeStruct((NP, D), jnp.float32),
    )(x, hist)


def _tc_mid(y, s, hist, c, W1, b1, W2):
    def body(y_ref, s_ref, h_ref, w1_ref, b1_ref, w2_ref, o_ref):
        dis = _dis_block(h_ref[...], c)
        t = (y_ref[...] + s_ref[0] + s_ref[1]) * dis
        x1 = jnp.dot(t, w1_ref[...], preferred_element_type=jnp.float32)
        x1 = jnp.maximum(x1 + b1_ref[...], 0.0)
        h2 = jnp.dot(x1, w2_ref[...], preferred_element_type=jnp.float32)
        o_ref[...] = h2 * dis

    return pl.pallas_call(
        body,
        grid=(NP // _BLK,),
        in_specs=[
            pl.BlockSpec((_BLK, D), lambda i: (i, 0)),
            pl.BlockSpec((NC, _BLK, D), lambda i: (0, i, 0)),
            pl.BlockSpec((NC, _BLK, D), lambda i: (0, i, 0)),
            pl.BlockSpec((D, DH), lambda i: (0, 0)),
            pl.BlockSpec((1, DH), lambda i: (0, 0)),
            pl.BlockSpec((DH, D), lambda i: (0, 0)),
        ],
        out_specs=pl.BlockSpec((_BLK, D), lambda i: (i, 0)),
        out_shape=jax.ShapeDtypeStruct((NP, D), jnp.float32),
    )(y, s, hist, W1, b1, W2)


def _tc_join(y, s, hist, b):
    def body(y_ref, s_ref, h_ref, b_ref, z_ref, yn_ref):
        dis_a = _dis_block(h_ref[...], 0)
        z = jnp.maximum((y_ref[...] + s_ref[0] + s_ref[1]) * dis_a + b_ref[...], 0.0)
        z_ref[...] = z
        yn_ref[...] = z * _dis_block(h_ref[...], 16)

    return pl.pallas_call(
        body,
        grid=(NP // _BLK,),
        in_specs=[
            pl.BlockSpec((_BLK, D), lambda i: (i, 0)),
            pl.BlockSpec((NC, _BLK, D), lambda i: (0, i, 0)),
            pl.BlockSpec((NC, _BLK, D), lambda i: (0, i, 0)),
            pl.BlockSpec((1, D), lambda i: (0, 0)),
        ],
        out_specs=[
            pl.BlockSpec((_BLK, D), lambda i: (i, 0)),
            pl.BlockSpec((_BLK, D), lambda i: (i, 0)),
        ],
        out_shape=[
            jax.ShapeDtypeStruct((NP, D), jnp.float32),
            jax.ShapeDtypeStruct((NP, D), jnp.float32),
        ],
    )(y, s, hist, b)


def _tc_post(y, s, hist, b):
    def body(y_ref, s_ref, h_ref, b_ref, z_ref):
        dis = _dis_block(h_ref[...], 16)
        z_ref[...] = jnp.maximum((y_ref[...] + s_ref[0] + s_ref[1]) * dis + b_ref[...], 0.0)

    return pl.pallas_call(
        body,
        grid=(NP // _BLK,),
        in_specs=[
            pl.BlockSpec((_BLK, D), lambda i: (i, 0)),
            pl.BlockSpec((NC, _BLK, D), lambda i: (0, i, 0)),
            pl.BlockSpec((NC, _BLK, D), lambda i: (0, i, 0)),
            pl.BlockSpec((1, D), lambda i: (0, 0)),
        ],
        out_specs=pl.BlockSpec((_BLK, D), lambda i: (i, 0)),
        out_shape=jax.ShapeDtypeStruct((NP, D), jnp.float32),
    )(y, s, hist, b)


def _prep_edges(ei):
    pad_dst = N + jnp.arange(EPAD - E, dtype=jnp.int32) % (NP - N)
    src = jnp.concatenate([ei[0], jnp.zeros((EPAD - E,), jnp.int32)])
    dst = jnp.concatenate([ei[1], pad_dst])
    return (src.reshape(EROWS, CHUNK), dst.reshape(EROWS, CHUNK),
            dst.reshape(HROWS, HCH))


def kernel(x, edge_index, edge_index2, W1, b1, W2, b2, Wg1, bg1, Wg2, bg2):
    ei1 = edge_index.astype(jnp.int32)
    ei2 = edge_index2.astype(jnp.int32)
    x_pad = jnp.pad(x, ((0, NP - N), (0, 0)))
    src1, dst1, hd1 = _prep_edges(ei1)
    src2, dst2, hd2 = _prep_edges(ei2)

    h = _sc_hist(hd1, hd2)

    y = _tc_scale(x_pad, h)
    s = _sc_aggregate(y, src1, dst1)
    y = _tc_mid(y, s, h, 0, W1, b1.reshape(1, DH), W2)
    s = _sc_aggregate(y, src1, dst1)
    z_pad, y = _tc_join(y, s, h, b2.reshape(1, D))

    s = _sc_aggregate(y, src2, dst2)
    y = _tc_mid(y, s, h, 16, Wg1, bg1.reshape(1, DH), Wg2)
    s = _sc_aggregate(y, src2, dst2)
    zg_pad = _tc_post(y, s, h, bg2.reshape(1, D))

    return z_pad[:N], zg_pad[:N]


# speedup vs baseline: 1.1901x; 1.1901x over previous
import functools

import jax
import jax.numpy as jnp
from jax import lax
from jax.experimental import pallas as pl
from jax.experimental.pallas import tpu as pltpu
from jax.experimental.pallas import tpu_sc as plsc

N = 10000
NP = 10240
D = 128
DH = 256
E = 320000
HCH = 128
HROWS = 2560
CHUNK = 64
EROWS = 5120
EPAD = 327680
NC, NS = 2, 16
NW = NC * NS
HRPT = HROWS // NW
RPT = EROWS // NW
HIB = 8
IB = 16
NBUF = 3
F0 = 4096
ZROWS = NP // NS

@functools.cache
def _mesh():
    return plsc.VectorSubcoreMesh(
        core_axis_name="c", subcore_axis_name="s", num_cores=NC, num_subcores=NS
    )


def _zero_vmem(ref):
    rows, cols = ref.shape

    @pl.loop(0, rows)
    def _(r):
        @pl.loop(0, cols, step=16)
        def _(c):
            ref[r, pl.ds(c, 16)] = jnp.zeros((16,), ref.dtype)


def _sc_hist(dsta, dstb):
    @functools.partial(
        pl.kernel,
        out_type=jax.ShapeDtypeStruct((NC, NP, D), jnp.float32),
        mesh=_mesh(),
        scratch_types=[
            pltpu.VMEM((HIB, HCH), jnp.int32),
            pltpu.VMEM((HCH, D), jnp.float32),
            pltpu.VMEM((HCH, D), jnp.float32),
            pltpu.VMEM((64, D), jnp.float32),
            pltpu.VMEM_SHARED((NP, D), jnp.float32),
        ],
    )
    def k(da_hbm, db_hbm, o_hbm, dstv, onea, oneb, zv, h):
        cid = lax.axis_index("c")
        sid = lax.axis_index("s")
        wid = cid * NS + sid
        e0 = jnp.where(
            lax.iota(jnp.int32, 16) == 0, jnp.float32(1.0), jnp.float32(0.0)
        )

        _zero_vmem(onea)
        _zero_vmem(oneb)
        _zero_vmem(zv)

        @pl.loop(0, HCH)
        def _(r):
            onea[r, pl.ds(0, 16)] = e0
            oneb[r, pl.ds(16, 16)] = e0

        @pl.loop(0, ZROWS, step=64)
        def _(i):
            pltpu.sync_copy(zv, h.at[pl.ds(sid * ZROWS + i, 64)])

        plsc.subcore_barrier()

        @pl.loop(0, HRPT, step=HIB)
        def _(base):
            pltpu.sync_copy(da_hbm.at[pl.ds(wid * HRPT + base, HIB)], dstv)

            @pl.loop(0, HIB)
            def _(j):
                pltpu.sync_copy(onea, h.at[dstv.at[j]], add=True)

        @pl.loop(0, HRPT, step=HIB)
        def _(base):
            pltpu.sync_copy(db_hbm.at[pl.ds(wid * HRPT + base, HIB)], dstv)

            @pl.loop(0, HIB)
            def _(j):
                pltpu.sync_copy(oneb, h.at[dstv.at[j]], add=True)

        plsc.subcore_barrier()
        pltpu.sync_copy(
            h.at[pl.ds(sid * ZROWS, ZROWS)],
            o_hbm.at[cid, pl.ds(sid * ZROWS, ZROWS)],
        )

    return k(dsta, dstb)


def _sc_aggregate(y, src2d, dst2d):
    @functools.partial(
        pl.kernel,
        out_type=jax.ShapeDtypeStruct((NC, NP, D), jnp.float32),
        mesh=_mesh(),
        scratch_types=[
            pltpu.VMEM((IB, CHUNK), jnp.int32),
            pltpu.VMEM((IB, CHUNK), jnp.int32),
            pltpu.VMEM((CHUNK, D), jnp.float32),
            pltpu.VMEM((CHUNK, D), jnp.float32),
            pltpu.VMEM((CHUNK, D), jnp.float32),
            pltpu.VMEM_SHARED((NP, D), jnp.float32),
            pltpu.SemaphoreType.DMA,
            pltpu.SemaphoreType.DMA,
            pltpu.SemaphoreType.DMA,
        ],
    )
    def k(y_hbm, src_hbm, dst_hbm, out_hbm,
          srcv, dstv, rb0, rb1, rb2, accum, s0, s1, s2):
        cid = lax.axis_index("c")
        sid = lax.axis_index("s")
        rbs = (rb0, rb1, rb2)
        sems = (s0, s1, s2)
        base = jnp.where(cid == 0, sid * (F0 // NS),
                         F0 + sid * ((EROWS - F0) // NS))
        nrows = jnp.where(cid == 0, F0 // NS, (EROWS - F0) // NS)

        _zero_vmem(rb0)

        @pl.loop(0, ZROWS, step=CHUNK)
        def _(i):
            pltpu.sync_copy(rb0, accum.at[pl.ds(sid * ZROWS + i, CHUNK)])

        plsc.subcore_barrier()

        @pl.loop(0, nrows, step=IB)
        def _(r):
            pltpu.sync_copy(src_hbm.at[pl.ds(base + r, IB)], srcv)
            pltpu.sync_copy(dst_hbm.at[pl.ds(base + r, IB)], dstv)
            for j in range(NBUF - 1):
                pltpu.async_copy(y_hbm.at[srcv.at[j]], rbs[j], sems[j])
            for j in range(IB):
                b = j % NBUF
                pltpu.make_async_copy(y_hbm.at[srcv.at[j]], rbs[b], sems[b]).wait()
                jn = j + NBUF - 1
                if jn < IB:
                    bn = jn % NBUF
                    pltpu.async_copy(y_hbm.at[srcv.at[jn]], rbs[bn], sems[bn])
                pltpu.sync_copy(rbs[b], accum.at[dstv.at[j]], add=True)

        plsc.subcore_barrier()
        pltpu.sync_copy(
            accum.at[pl.ds(sid * ZROWS, ZROWS)],
            out_hbm.at[cid, pl.ds(sid * ZROWS, ZROWS)],
        )

    return k(y, src2d, dst2d)


_BLK = 1024


def _dis_block(h, c):
    deg = 1.0 + h[0, :, c:c + 1] + h[1, :, c:c + 1]
    return lax.rsqrt(deg)


def _tc_scale(x, hist):
    def body(x_ref, h_ref, o_ref):
        o_ref[...] = x_ref[...] * _dis_block(h_ref[...], 0)

    return pl.pallas_call(
        body,
        grid=(NP // _BLK,),
        in_specs=[
            pl.BlockSpec((_BLK, D), lambda i: (i, 0)),
            pl.BlockSpec((NC, _BLK, D), lambda i: (0, i, 0)),
        ],
        out_specs=pl.BlockSpec((_BLK, D), lambda i: (i, 0)),
        out_shape=jax.ShapeDtypeStruct((NP, D), jnp.float32),
    )(x, hist)


def _tc_mid(y, s, hist, c, W1, b1, W2):
    def body(y_ref, s_ref, h_ref, w1_ref, b1_ref, w2_ref, o_ref):
        dis = _dis_block(h_ref[...], c)
        t = (y_ref[...] + s_ref[0] + s_ref[1]) * dis
        x1 = jnp.dot(t, w1_ref[...], preferred_element_type=jnp.float32)
        x1 = jnp.maximum(x1 + b1_ref[...], 0.0)
        h2 = jnp.dot(x1, w2_ref[...], preferred_element_type=jnp.float32)
        o_ref[...] = h2 * dis

    return pl.pallas_call(
        body,
        grid=(NP // _BLK,),
        in_specs=[
            pl.BlockSpec((_BLK, D), lambda i: (i, 0)),
            pl.BlockSpec((NC, _BLK, D), lambda i: (0, i, 0)),
            pl.BlockSpec((NC, _BLK, D), lambda i: (0, i, 0)),
            pl.BlockSpec((D, DH), lambda i: (0, 0)),
            pl.BlockSpec((1, DH), lambda i: (0, 0)),
            pl.BlockSpec((DH, D), lambda i: (0, 0)),
        ],
        out_specs=pl.BlockSpec((_BLK, D), lambda i: (i, 0)),
        out_shape=jax.ShapeDtypeStruct((NP, D), jnp.float32),
    )(y, s, hist, W1, b1, W2)


def _tc_join(y, s, hist, b):
    def body(y_ref, s_ref, h_ref, b_ref, z_ref, yn_ref):
        dis_a = _dis_block(h_ref[...], 0)
        z = jnp.maximum((y_ref[...] + s_ref[0] + s_ref[1]) * dis_a + b_ref[...], 0.0)
        z_ref[...] = z
        yn_ref[...] = z * _dis_block(h_ref[...], 16)

    return pl.pallas_call(
        body,
        grid=(NP // _BLK,),
        in_specs=[
            pl.BlockSpec((_BLK, D), lambda i: (i, 0)),
            pl.BlockSpec((NC, _BLK, D), lambda i: (0, i, 0)),
            pl.BlockSpec((NC, _BLK, D), lambda i: (0, i, 0)),
            pl.BlockSpec((1, D), lambda i: (0, 0)),
        ],
        out_specs=[
            pl.BlockSpec((_BLK, D), lambda i: (i, 0)),
            pl.BlockSpec((_BLK, D), lambda i: (i, 0)),
        ],
        out_shape=[
            jax.ShapeDtypeStruct((NP, D), jnp.float32),
            jax.ShapeDtypeStruct((NP, D), jnp.float32),
        ],
    )(y, s, hist, b)


def _tc_post(y, s, hist, b):
    def body(y_ref, s_ref, h_ref, b_ref, z_ref):
        dis = _dis_block(h_ref[...], 16)
        z_ref[...] = jnp.maximum((y_ref[...] + s_ref[0] + s_ref[1]) * dis + b_ref[...], 0.0)

    return pl.pallas_call(
        body,
        grid=(NP // _BLK,),
        in_specs=[
            pl.BlockSpec((_BLK, D), lambda i: (i, 0)),
            pl.BlockSpec((NC, _BLK, D), lambda i: (0, i, 0)),
            pl.BlockSpec((NC, _BLK, D), lambda i: (0, i, 0)),
            pl.BlockSpec((1, D), lambda i: (0, 0)),
        ],
        out_specs=pl.BlockSpec((_BLK, D), lambda i: (i, 0)),
        out_shape=jax.ShapeDtypeStruct((NP, D), jnp.float32),
    )(y, s, hist, b)


def _prep_edges(ei):
    pad_dst = N + jnp.arange(EPAD - E, dtype=jnp.int32) % (NP - N)
    src = jnp.concatenate([ei[0], jnp.zeros((EPAD - E,), jnp.int32)])
    dst = jnp.concatenate([ei[1], pad_dst])
    return (src.reshape(EROWS, CHUNK), dst.reshape(EROWS, CHUNK),
            dst.reshape(HROWS, HCH))


def kernel(x, edge_index, edge_index2, W1, b1, W2, b2, Wg1, bg1, Wg2, bg2):
    ei1 = edge_index.astype(jnp.int32)
    ei2 = edge_index2.astype(jnp.int32)
    x_pad = jnp.pad(x, ((0, NP - N), (0, 0)))
    src1, dst1, hd1 = _prep_edges(ei1)
    src2, dst2, hd2 = _prep_edges(ei2)

    h = _sc_hist(hd1, hd2)

    y = _tc_scale(x_pad, h)
    s = _sc_aggregate(y, src1, dst1)
    y = _tc_mid(y, s, h, 0, W1, b1.reshape(1, DH), W2)
    s = _sc_aggregate(y, src1, dst1)
    z_pad, y = _tc_join(y, s, h, b2.reshape(1, D))

    s = _sc_aggregate(y, src2, dst2)
    y = _tc_mid(y, s, h, 16, Wg1, bg1.reshape(1, DH), Wg2)
    s = _sc_aggregate(y, src2, dst2)
    zg_pad = _tc_post(y, s, h, bg2.reshape(1, D))

    return z_pad[:N], zg_pad[:N]
